# Initial kernel scaffold; baseline (speedup 1.0000x reference)
import jax
import jax.numpy as jnp
from jax.experimental import pallas as pl


def kernel(edge_index, edge_weight, batch, W1, b1, W2, b2, W3, b3, W4, b4, W5, b5, lin_w, lin_b):
    raise NotImplementedError("write your pallas kernel here")


# speedup vs baseline: 3.5246x; 3.5246x over previous
import functools

import jax
import jax.numpy as jnp
from jax import lax
from jax.experimental import pallas as pl
from jax.experimental.pallas import tpu as pltpu
from jax.experimental.pallas import tpu_sc as plsc

_NN = 100000
_EE = 1600000
_HH = 32
_FH = 16
_GG = 128

NP = 102400
EP = 1638400
HALF = NP // 2
ACC_ROWS = HALF + 8
SUP = 2048
SUB = 128
NSUB = SUP // SUB
NSUP_MM = (EP // 16) // SUP
NSUP_DG = (EP // 32) // SUP
RPT = HALF // 16
DPT = NP // 16
NPT = NP // 32
NCH_PL = NPT // SUB

_mesh = functools.partial(
    plsc.VectorSubcoreMesh, core_axis_name="c", subcore_axis_name="s"
)
_sc_params = pltpu.CompilerParams(use_tc_tiling_on_sc=False)


def _f32(*shape):
    return jax.ShapeDtypeStruct(shape, jnp.float32)


def _deg_body(dst2d, ew, zeros1d, out, dst_v, ew_v, acc, sem):
    c = lax.axis_index("c")
    s = lax.axis_index("s")
    w = c * 16 + s
    pltpu.sync_copy(zeros1d, acc.at[pl.ds(s * DPT, DPT)])
    plsc.subcore_barrier()

    def sup(i, _):
        eb = pl.multiple_of(w * (EP // 32) + i * SUP, SUP)
        rb = pl.multiple_of(eb // SUB, NSUB)
        pltpu.sync_copy(dst2d.at[pl.ds(rb, NSUB)], dst_v)
        pltpu.sync_copy(ew.at[pl.ds(eb, SUP)], ew_v)
        cps = []
        for j in range(NSUB):
            cps.append(
                pltpu.async_copy(
                    ew_v.at[pl.ds(j * SUB, SUB)], acc.at[dst_v.at[j]], sem,
                    add=True,
                )
            )
        for cp in cps:
            cp.wait()
        return 0

    lax.fori_loop(0, NSUP_DG, sup, 0)
    plsc.subcore_barrier()
    pltpu.sync_copy(acc.at[pl.ds(s * DPT, DPT)], out.at[c, pl.ds(s * DPT, DPT)])


_deg_kernel = functools.partial(
    pl.kernel,
    out_type=_f32(2, NP),
    mesh=_mesh(),
    scratch_types=[
        pltpu.VMEM((NSUB, SUB), jnp.int32),
        pltpu.VMEM((SUP,), jnp.float32),
        pltpu.VMEM_SHARED((NP,), jnp.float32),
        pltpu.SemaphoreType.DMA,
    ],
    compiler_params=_sc_params,
)(_deg_body)


def _spmm_body(x0, x1, src, dst2d, ew, zeros2d, agg0, agg1,
               src_v, dst_v, ew_v, rows_v, acc, sem_g, sem_s):
    c = lax.axis_index("c")
    s = lax.axis_index("s")
    base_row = c * HALF

    for p in range(2):
        xp = (x0, x1)[p]
        aggp = (agg0, agg1)[p]
        pltpu.sync_copy(zeros2d, acc.at[pl.ds(s * RPT, RPT)])
        plsc.subcore_barrier()

        def sup(i, _):
            eb = pl.multiple_of(s * (EP // 16) + i * SUP, SUP)
            rb = pl.multiple_of(eb // SUB, NSUB)
            pltpu.sync_copy(src.at[pl.ds(eb, SUP)], src_v)
            pltpu.sync_copy(dst2d.at[pl.ds(rb, NSUB)], dst_v)
            pltpu.sync_copy(ew.at[pl.ds(eb, SUP)], ew_v)
            gcps = []
            for j in range(NSUB):
                gcps.append(
                    pltpu.async_copy(
                        xp.at[src_v.at[pl.ds(j * SUB, SUB)]], rows_v.at[j],
                        sem_g,
                    )
                )

            def adj(j, _):
                for g in range(SUB // 16):
                    v = dst_v[j, pl.ds(g * 16, 16)]
                    lv = v - base_row
                    ok = (lv >= 0) & (lv < HALF)
                    dst_v[j, pl.ds(g * 16, 16)] = jnp.where(ok, lv, HALF)
                return 0

            lax.fori_loop(0, NSUB, adj, 0)
            for cp in gcps:
                cp.wait()

            dnums = lax.GatherDimensionNumbers(
                offset_dims=(), collapsed_slice_dims=(0,),
                start_index_map=(0,),
            )

            def scale(g, _):
                j = g >> 3
                e0 = (g & 7) * 16
                ewg = ew_v[pl.ds(g * 16, 16)]
                for jj in range(16):
                    ewbc = lax.gather(
                        ewg, jnp.full((16, 1), jj, jnp.int32), dnums,
                        slice_sizes=(1,),
                        mode=lax.GatherScatterMode.PROMISE_IN_BOUNDS,
                    )
                    e = e0 + jj
                    rows_v[j, e, pl.ds(0, _FH)] = (
                        rows_v[j, e, pl.ds(0, _FH)] * ewbc
                    )
                return 0

            lax.fori_loop(0, SUP // 16, scale, 0)

            scps = []
            for j in range(NSUB):
                scps.append(
                    pltpu.async_copy(
                        rows_v.at[j], acc.at[dst_v.at[j]], sem_s, add=True
                    )
                )
            for cp in scps:
                cp.wait()
            return 0

        lax.fori_loop(0, NSUP_MM, sup, 0)
        plsc.subcore_barrier()
        pltpu.sync_copy(
            acc.at[pl.ds(s * RPT, RPT)],
            aggp.at[pl.ds(base_row + s * RPT, RPT)],
        )


_spmm_kernel = functools.partial(
    pl.kernel,
    out_type=(_f32(NP, _FH), _f32(NP, _FH)),
    mesh=_mesh(),
    scratch_types=[
        pltpu.VMEM((SUP,), jnp.int32),
        pltpu.VMEM((NSUB, SUB), jnp.int32),
        pltpu.VMEM((SUP,), jnp.float32),
        pltpu.VMEM((NSUB, SUB, _FH), jnp.float32),
        pltpu.VMEM_SHARED((ACC_ROWS, _FH), jnp.float32),
        pltpu.SemaphoreType.DMA,
        pltpu.SemaphoreType.DMA,
    ],
    compiler_params=_sc_params,
)(_spmm_body)


def _pool_body(x0, x1, bat2d, zeros2d, zeros1d, sums0, sums1, cnt,
               xin_v, bt_v, ones_v, accs0, accs1, accc, sem):
    c = lax.axis_index("c")
    s = lax.axis_index("s")
    w = c * 16 + s

    @pl.when(s == 0)
    def _():
        pltpu.sync_copy(zeros2d.at[pl.ds(0, _GG + 8)], accs0)
        pltpu.sync_copy(zeros2d.at[pl.ds(0, _GG + 8)], accs1)
        pltpu.sync_copy(zeros1d.at[pl.ds(0, _GG + 8)], accc)

    def ones_fill(i, _):
        ones_v[pl.ds(i * 16, 16)] = jnp.zeros((16,), jnp.float32) + 1.0
        return 0

    lax.fori_loop(0, SUB // 16, ones_fill, 0)
    plsc.subcore_barrier()

    def chunk(t, _):
        nb = pl.multiple_of(w * NPT + t * SUB, SUB)
        pltpu.sync_copy(bat2d.at[nb // SUB], bt_v)
        pltpu.sync_copy(x0.at[pl.ds(nb, SUB)], xin_v)
        pltpu.sync_copy(xin_v, accs0.at[bt_v], add=True)
        pltpu.sync_copy(x1.at[pl.ds(nb, SUB)], xin_v)
        pltpu.sync_copy(xin_v, accs1.at[bt_v], add=True)
        pltpu.sync_copy(ones_v, accc.at[bt_v], add=True)
        return 0

    lax.fori_loop(0, NCH_PL, chunk, 0)
    plsc.subcore_barrier()

    @pl.when(s == 0)
    def _():
        pltpu.sync_copy(accs0.at[pl.ds(0, _GG)], sums0.at[c])
        pltpu.sync_copy(accs1.at[pl.ds(0, _GG)], sums1.at[c])
        pltpu.sync_copy(accc.at[pl.ds(0, _GG)], cnt.at[c])


_pool_kernel = functools.partial(
    pl.kernel,
    out_type=(_f32(2, _GG, _FH), _f32(2, _GG, _FH), _f32(2, _GG)),
    mesh=_mesh(),
    scratch_types=[
        pltpu.VMEM((SUB, _FH), jnp.float32),
        pltpu.VMEM((SUB,), jnp.int32),
        pltpu.VMEM((SUB,), jnp.float32),
        pltpu.VMEM_SHARED((_GG + 8, _FH), jnp.float32),
        pltpu.VMEM_SHARED((_GG + 8, _FH), jnp.float32),
        pltpu.VMEM_SHARED((_GG + 8,), jnp.float32),
        pltpu.SemaphoreType.DMA,
    ],
    compiler_params=_sc_params,
)(_pool_body)


_BLK = 6400


def _lay1_body(dg_ref, w_ref, b_ref, o0_ref, o1_ref):
    d = dg_ref[0, :] + dg_ref[1, :]
    y = d[:, None] * w_ref[0, :][None, :] + b_ref[0, :][None, :]
    y = jnp.maximum(y, 0.0)
    o0_ref[...] = y[:, 0:_FH]
    o1_ref[...] = y[:, _FH:_HH]


def _layer1(dgw_p, w1row, b1row):
    return pl.pallas_call(
        _lay1_body,
        out_shape=(_f32(NP, _FH), _f32(NP, _FH)),
        grid=(NP // _BLK,),
        in_specs=[
            pl.BlockSpec((2, _BLK), lambda i: (0, i)),
            pl.BlockSpec((1, _HH), lambda i: (0, 0)),
            pl.BlockSpec((1, _HH), lambda i: (0, 0)),
        ],
        out_specs=(
            pl.BlockSpec((_BLK, _FH), lambda i: (i, 0)),
            pl.BlockSpec((_BLK, _FH), lambda i: (i, 0)),
        ),
    )(dgw_p, w1row, b1row)


def _mm_body(a0_ref, a1_ref, wt_ref, b_ref, o0_ref, o1_ref):
    y = (
        jnp.dot(a0_ref[...], wt_ref[0:_FH, :],
                preferred_element_type=jnp.float32)
        + jnp.dot(a1_ref[...], wt_ref[_FH:_HH, :],
                  preferred_element_type=jnp.float32)
        + b_ref[0, :][None, :]
    )
    y = jnp.maximum(y, 0.0)
    o0_ref[...] = y[:, 0:_FH]
    o1_ref[...] = y[:, _FH:_HH]


def _mm_relu(a0, a1, wt, brow):
    return pl.pallas_call(
        _mm_body,
        out_shape=(_f32(NP, _FH), _f32(NP, _FH)),
        grid=(NP // _BLK,),
        in_specs=[
            pl.BlockSpec((_BLK, _FH), lambda i: (i, 0)),
            pl.BlockSpec((_BLK, _FH), lambda i: (i, 0)),
            pl.BlockSpec((_HH, _HH), lambda i: (0, 0)),
            pl.BlockSpec((1, _HH), lambda i: (0, 0)),
        ],
        out_specs=(
            pl.BlockSpec((_BLK, _FH), lambda i: (i, 0)),
            pl.BlockSpec((_BLK, _FH), lambda i: (i, 0)),
        ),
    )(a0, a1, wt, brow)


def _head_body(s0_ref, s1_ref, c_ref, lw_ref, lb_ref, o_ref):
    sm0 = s0_ref[0:_GG, :] + s0_ref[_GG : 2 * _GG, :]
    sm1 = s1_ref[0:_GG, :] + s1_ref[_GG : 2 * _GG, :]
    ct = jnp.maximum(c_ref[0, :] + c_ref[1, :], 1.0)[:, None]
    o_ref[...] = (
        jnp.dot(sm0 / ct, lw_ref[0:_FH, :],
                preferred_element_type=jnp.float32)
        + jnp.dot(sm1 / ct, lw_ref[_FH:_HH, :],
                  preferred_element_type=jnp.float32)
        + lb_ref[0, 0]
    )


def _head(sums0_p, sums1_p, cnt_p, lin_wt, lin_b2):
    return pl.pallas_call(
        _head_body,
        out_shape=_f32(_GG, 1),
        in_specs=[
            pl.BlockSpec((2 * _GG, _FH), lambda: (0, 0)),
            pl.BlockSpec((2 * _GG, _FH), lambda: (0, 0)),
            pl.BlockSpec((2, _GG), lambda: (0, 0)),
            pl.BlockSpec((_HH, 1), lambda: (0, 0)),
            pl.BlockSpec((1, 1), lambda: (0, 0)),
        ],
        out_specs=pl.BlockSpec((_GG, 1), lambda: (0, 0)),
    )(
        sums0_p.reshape(2 * _GG, _FH),
        sums1_p.reshape(2 * _GG, _FH),
        cnt_p,
        lin_wt,
        lin_b2,
    )


def kernel(edge_index, edge_weight, batch, W1, b1, W2, b2, W3, b3, W4, b4,
           W5, b5, lin_w, lin_b):
    src = edge_index[0].astype(jnp.int32)
    dst = edge_index[1].astype(jnp.int32)
    ew = edge_weight.astype(jnp.float32)
    pad_e = EP - _EE
    srcp = jnp.concatenate([src, jnp.zeros((pad_e,), jnp.int32)])
    dstp = jnp.concatenate([dst, jnp.zeros((pad_e,), jnp.int32)])
    ewp = jnp.concatenate([ew, jnp.zeros((pad_e,), jnp.float32)])
    dst2d = dstp.reshape(EP // SUB, SUB)
    batp = jnp.concatenate(
        [batch.astype(jnp.int32), jnp.full((NP - _NN,), _GG, jnp.int32)]
    )
    bat2d = batp.reshape(NP // SUB, SUB)
    zeros2d = jnp.zeros((RPT, _FH), jnp.float32)
    zeros1d = jnp.zeros((DPT,), jnp.float32)

    dgw_p = _deg_kernel(dst2d, ewp, zeros1d)
    x0, x1 = _layer1(dgw_p, W1.reshape(1, _HH), b1.reshape(1, _HH))
    for Wl, bl in ((W2, b2), (W3, b3), (W4, b4), (W5, b5)):
        a0, a1 = _spmm_kernel(x0, x1, srcp, dst2d, ewp, zeros2d)
        x0, x1 = _mm_relu(a0, a1, Wl.T, bl.reshape(1, _HH))
    sums0_p, sums1_p, cnt_p = _pool_kernel(x0, x1, bat2d, zeros2d, zeros1d)
    return _head(sums0_p, sums1_p, cnt_p, lin_w.T, lin_b.reshape(1, 1))
